# Initial kernel scaffold; baseline (speedup 1.0000x reference)
import jax
import jax.numpy as jnp
from jax import lax
from jax.experimental import pallas as pl
from jax.experimental.pallas import tpu as pltpu
from jax.experimental.pallas import tpu_sc as plsc

N = 10000
E = 320000
D = 128
NC = 2
NS = 16
NW = NC * NS
EPT = E // NW
B = 80
NB = EPT // B
L = 16
NPAD = 10240
STRIPE = NPAD // NS

_mesh = plsc.VectorSubcoreMesh(
    core_axis_name="c", subcore_axis_name="s", num_cores=NC, num_subcores=NS
)


def _proj_body(x_ref, w_ref, as_ref, ad_ref, h_ref, als_ref, ald_ref):
    h = jnp.dot(x_ref[...], w_ref[...], preferred_element_type=jnp.float32)
    h_ref[...] = h
    als_ref[...] = jnp.sum(h * as_ref[...], axis=1, keepdims=True)
    ald_ref[...] = jnp.sum(h * ad_ref[...], axis=1, keepdims=True)


def _proj(x, W, a_src, a_dst):
    h, als, ald = pl.pallas_call(
        _proj_body,
        out_shape=[
            jax.ShapeDtypeStruct((N, D), jnp.float32),
            jax.ShapeDtypeStruct((N, 1), jnp.float32),
            jax.ShapeDtypeStruct((N, 1), jnp.float32),
        ],
    )(x, W, a_src.reshape(1, D), a_dst.reshape(1, D))
    return h, als.reshape(N), ald.reshape(N)


def _bn(t, g_ref, be_ref):
    mu = jnp.mean(t, axis=0, keepdims=True)
    xm = t - mu
    var = jnp.mean(xm * xm, axis=0, keepdims=True)
    return xm * lax.rsqrt(var + 1e-5) * g_ref[...] + be_ref[...]


def _mid_body(p0_ref, p1_ref, b_ref, g_ref, be_ref, w_ref, as_ref, ad_ref,
              h_ref, als_ref, ald_ref):
    t = p0_ref[0:N, :] + p1_ref[0:N, :] + b_ref[...]
    y = jnp.maximum(_bn(t, g_ref, be_ref), 0.0)
    h = jnp.dot(y, w_ref[...], preferred_element_type=jnp.float32)
    h_ref[...] = h
    als_ref[...] = jnp.sum(h * as_ref[...], axis=1, keepdims=True)
    ald_ref[...] = jnp.sum(h * ad_ref[...], axis=1, keepdims=True)


def _mid(p0, p1, b, g, be, W, a_src, a_dst):
    h, als, ald = pl.pallas_call(
        _mid_body,
        out_shape=[
            jax.ShapeDtypeStruct((N, D), jnp.float32),
            jax.ShapeDtypeStruct((N, 1), jnp.float32),
            jax.ShapeDtypeStruct((N, 1), jnp.float32),
        ],
    )(p0, p1, b.reshape(1, D), g.reshape(1, D), be.reshape(1, D),
      W, a_src.reshape(1, D), a_dst.reshape(1, D))
    return h, als.reshape(N), ald.reshape(N)


def _final_body(p0_ref, p1_ref, b_ref, g_ref, be_ref, out_ref):
    t = p0_ref[0:N, :] + p1_ref[0:N, :] + b_ref[...]
    out_ref[...] = _bn(t, g_ref, be_ref)


def _final(p0, p1, b, g, be):
    return pl.pallas_call(
        _final_body,
        out_shape=jax.ShapeDtypeStruct((N, D), jnp.float32),
    )(p0, p1, b.reshape(1, D), g.reshape(1, D), be.reshape(1, D))


def _edge_softmax_body(src_h, dst_h, als_h, ald_h, s_out, ex_out,
                       als_v, ald_v, srcb, dstb, exb, s_loc, accv, tmpv,
                       s_stage):
    cid = lax.axis_index("c")
    sid = lax.axis_index("s")
    wid = cid * NS + sid
    pltpu.sync_copy(als_h, als_v)
    pltpu.sync_copy(ald_h, ald_v)
    pltpu.sync_copy(src_h.at[wid], srcb)
    pltpu.sync_copy(dst_h.at[wid], dstb)

    z16 = jnp.zeros((L,), jnp.float32)

    def _zero(i, carry):
        s_loc[pl.ds(i * L, L)] = z16
        return carry

    lax.fori_loop(0, NPAD // L, _zero, 0)

    def _row(j, carry):
        for k in range(B // L):
            s16 = srcb[j, pl.ds(k * L, L)]
            d16 = dstb[j, pl.ds(k * L, L)]
            e = plsc.load_gather(als_v, [s16]) + plsc.load_gather(ald_v, [d16])
            e = jnp.where(e > 0.0, e, 0.2 * e)
            ex = jnp.exp(e)
            exb[j, pl.ds(k * L, L)] = ex
            plsc.addupdate_scatter(s_loc, [d16], ex)
        return carry

    lax.fori_loop(0, NB, _row, 0)
    pltpu.sync_copy(exb, ex_out.at[wid])
    pltpu.sync_copy(s_loc, s_stage.at[sid])
    plsc.subcore_barrier()

    base = sid * STRIPE
    pltpu.sync_copy(s_stage.at[0, pl.ds(base, STRIPE)], accv)
    for t in range(1, NS):
        pltpu.sync_copy(s_stage.at[t, pl.ds(base, STRIPE)], tmpv)

        def _acc(i, carry):
            accv[pl.ds(i * L, L)] = (
                accv[pl.ds(i * L, L)] + tmpv[pl.ds(i * L, L)]
            )
            return carry

        lax.fori_loop(0, STRIPE // L, _acc, 0)
    pltpu.sync_copy(accv, s_out.at[cid, pl.ds(base, STRIPE)])


def _edge_softmax(src3, dst3, als, ald):
    return pl.kernel(
        _edge_softmax_body,
        out_type=[
            jax.ShapeDtypeStruct((NC, NPAD), jnp.float32),
            jax.ShapeDtypeStruct((NW, NB, B), jnp.float32),
        ],
        mesh=_mesh,
        scratch_types=[
            pltpu.VMEM((N,), jnp.float32),
            pltpu.VMEM((N,), jnp.float32),
            pltpu.VMEM((NB, B), jnp.int32),
            pltpu.VMEM((NB, B), jnp.int32),
            pltpu.VMEM((NB, B), jnp.float32),
            pltpu.VMEM((NPAD,), jnp.float32),
            pltpu.VMEM((STRIPE,), jnp.float32),
            pltpu.VMEM((STRIPE,), jnp.float32),
            pltpu.VMEM_SHARED((NS, NPAD), jnp.float32),
        ],
    )(src3, dst3, als, ald)


def _aggregate_body(src_h, dst_h, ex_h, s_h, feat_h, out_h,
                    s_comb, tmp_s, srcb, dstb, exb, alpha_v, rows, zrows,
                    out_sh, sem):
    cid = lax.axis_index("c")
    sid = lax.axis_index("s")
    wid = cid * NS + sid
    pltpu.sync_copy(s_h.at[0], s_comb)
    pltpu.sync_copy(s_h.at[1], tmp_s)

    def _add(i, carry):
        s_comb[pl.ds(i * L, L)] = (
            s_comb[pl.ds(i * L, L)] + tmp_s[pl.ds(i * L, L)]
        )
        return carry

    lax.fori_loop(0, NPAD // L, _add, 0)
    pltpu.sync_copy(src_h.at[wid], srcb)
    pltpu.sync_copy(dst_h.at[wid], dstb)
    pltpu.sync_copy(ex_h.at[wid], exb)

    z16 = jnp.zeros((L,), jnp.float32)

    def _zero(i, carry):
        for k in range(D // L):
            zrows[i, pl.ds(k * L, L)] = z16
        return carry

    lax.fori_loop(0, B, _zero, 0)
    for i in range(STRIPE // B):
        pltpu.sync_copy(zrows, out_sh.at[pl.ds(sid * STRIPE + i * B, B)])
    plsc.subcore_barrier()

    def _row(j, carry):
        pltpu.async_copy(feat_h.at[srcb.at[j]], rows, sem).wait()
        for k in range(B // L):
            d16 = dstb[j, pl.ds(k * L, L)]
            s16 = plsc.load_gather(s_comb, [d16])
            ex16 = exb[j, pl.ds(k * L, L)]
            alpha_v[pl.ds(k * L, L)] = ex16 / (s16 + 1e-16)

        def _edge(e2, c2):
            asp = plsc.load_gather(alpha_v, [jnp.zeros((L,), jnp.int32) + e2])
            for k in range(D // L):
                rows[e2, pl.ds(k * L, L)] = rows[e2, pl.ds(k * L, L)] * asp
            return c2

        lax.fori_loop(0, B, _edge, 0)
        pltpu.sync_copy(rows, out_sh.at[dstb.at[j]], add=True)
        return carry

    lax.fori_loop(0, NB, _row, 0)
    plsc.subcore_barrier()
    pltpu.sync_copy(
        out_sh.at[pl.ds(sid * STRIPE, STRIPE)],
        out_h.at[cid, pl.ds(sid * STRIPE, STRIPE)],
    )


def _aggregate(src3, dst3, ex3, s_parts, feat):
    return pl.kernel(
        _aggregate_body,
        out_type=jax.ShapeDtypeStruct((NC, NPAD, D), jnp.float32),
        mesh=_mesh,
        scratch_types=[
            pltpu.VMEM((NPAD,), jnp.float32),
            pltpu.VMEM((NPAD,), jnp.float32),
            pltpu.VMEM((NB, B), jnp.int32),
            pltpu.VMEM((NB, B), jnp.int32),
            pltpu.VMEM((NB, B), jnp.float32),
            pltpu.VMEM((B,), jnp.float32),
            pltpu.VMEM((B, D), jnp.float32),
            pltpu.VMEM((B, D), jnp.float32),
            pltpu.VMEM_SHARED((NPAD, D), jnp.float32),
            pltpu.SemaphoreType.DMA,
        ],
    )(src3, dst3, ex3, s_parts, feat)


def kernel(x, edge_index, W1, a_src1, a_dst1, b1, g1, be1,
           W2, a_src2, a_dst2, b2, g2, be2):
    src3 = edge_index[0].reshape(NW, NB, B)
    dst3 = edge_index[1].reshape(NW, NB, B)
    h1, als1, ald1 = _proj(x, W1, a_src1, a_dst1)
    s1, ex1 = _edge_softmax(src3, dst3, als1, ald1)
    p1 = _aggregate(src3, dst3, ex1, s1, h1)
    h2, als2, ald2 = _mid(p1[0], p1[1], b1, g1, be1, W2, a_src2, a_dst2)
    s2, ex2 = _edge_softmax(src3, dst3, als2, ald2)
    p2 = _aggregate(src3, dst3, ex2, s2, h2)
    return _final(p2[0], p2[1], b2, g2, be2)


# speedup vs baseline: 25.6373x; 25.6373x over previous
import jax
import jax.numpy as jnp
from jax import lax
from jax.experimental import pallas as pl
from jax.experimental.pallas import tpu as pltpu
from jax.experimental.pallas import tpu_sc as plsc

N = 10000
E = 320000
D = 128
NC = 2
NS = 16
NW = NC * NS
EPT = E // NW
B = 80
NB = EPT // B
L = 16
NPAD = 10240
STRIPE = NPAD // NS

_mesh = plsc.VectorSubcoreMesh(
    core_axis_name="c", subcore_axis_name="s", num_cores=NC, num_subcores=NS
)


def _proj_body(x_ref, w_ref, as_ref, ad_ref, h_ref, als_ref, ald_ref):
    h = jnp.dot(x_ref[...], w_ref[...], preferred_element_type=jnp.float32)
    h_ref[...] = h
    als_ref[...] = jnp.sum(h * as_ref[...], axis=1, keepdims=True)
    ald_ref[...] = jnp.sum(h * ad_ref[...], axis=1, keepdims=True)


def _proj(x, W, a_src, a_dst):
    h, als, ald = pl.pallas_call(
        _proj_body,
        out_shape=[
            jax.ShapeDtypeStruct((N, D), jnp.float32),
            jax.ShapeDtypeStruct((N, 1), jnp.float32),
            jax.ShapeDtypeStruct((N, 1), jnp.float32),
        ],
    )(x, W, a_src.reshape(1, D), a_dst.reshape(1, D))
    return h, als.reshape(N), ald.reshape(N)


def _bn(t, g_ref, be_ref):
    mu = jnp.mean(t, axis=0, keepdims=True)
    xm = t - mu
    var = jnp.mean(xm * xm, axis=0, keepdims=True)
    return xm * lax.rsqrt(var + 1e-5) * g_ref[...] + be_ref[...]


def _mid_body(p0_ref, p1_ref, b_ref, g_ref, be_ref, w_ref, as_ref, ad_ref,
              h_ref, als_ref, ald_ref):
    t = p0_ref[0:N, :] + p1_ref[0:N, :] + b_ref[...]
    y = jnp.maximum(_bn(t, g_ref, be_ref), 0.0)
    h = jnp.dot(y, w_ref[...], preferred_element_type=jnp.float32)
    h_ref[...] = h
    als_ref[...] = jnp.sum(h * as_ref[...], axis=1, keepdims=True)
    ald_ref[...] = jnp.sum(h * ad_ref[...], axis=1, keepdims=True)


def _mid(p0, p1, b, g, be, W, a_src, a_dst):
    h, als, ald = pl.pallas_call(
        _mid_body,
        out_shape=[
            jax.ShapeDtypeStruct((N, D), jnp.float32),
            jax.ShapeDtypeStruct((N, 1), jnp.float32),
            jax.ShapeDtypeStruct((N, 1), jnp.float32),
        ],
    )(p0, p1, b.reshape(1, D), g.reshape(1, D), be.reshape(1, D),
      W, a_src.reshape(1, D), a_dst.reshape(1, D))
    return h, als.reshape(N), ald.reshape(N)


def _final_body(p0_ref, p1_ref, b_ref, g_ref, be_ref, out_ref):
    t = p0_ref[0:N, :] + p1_ref[0:N, :] + b_ref[...]
    out_ref[...] = _bn(t, g_ref, be_ref)


def _final(p0, p1, b, g, be):
    return pl.pallas_call(
        _final_body,
        out_shape=jax.ShapeDtypeStruct((N, D), jnp.float32),
    )(p0, p1, b.reshape(1, D), g.reshape(1, D), be.reshape(1, D))


def _edge_softmax_body(src_h, dst_h, als_h, ald_h, s_out, ex_out,
                       als_v, ald_v, srcb, dstb, exb, s_loc, accv, tmpv,
                       s_stage):
    cid = lax.axis_index("c")
    sid = lax.axis_index("s")
    wid = cid * NS + sid
    pltpu.sync_copy(als_h, als_v)
    pltpu.sync_copy(ald_h, ald_v)
    pltpu.sync_copy(src_h.at[wid], srcb)
    pltpu.sync_copy(dst_h.at[wid], dstb)

    z16 = jnp.zeros((L,), jnp.float32)

    def _zero(i, carry):
        s_loc[pl.ds(i * L, L)] = z16
        return carry

    lax.fori_loop(0, NPAD // L, _zero, 0)

    def _row(j, carry):
        for k in range(B // L):
            s16 = srcb[j, pl.ds(k * L, L)]
            d16 = dstb[j, pl.ds(k * L, L)]
            e = plsc.load_gather(als_v, [s16]) + plsc.load_gather(ald_v, [d16])
            e = jnp.where(e > 0.0, e, 0.2 * e)
            ex = jnp.exp(e)
            exb[j, pl.ds(k * L, L)] = ex
            plsc.addupdate_scatter(s_loc, [d16], ex)
        return carry

    lax.fori_loop(0, NB, _row, 0)
    pltpu.sync_copy(exb, ex_out.at[wid])
    pltpu.sync_copy(s_loc, s_stage.at[sid])
    plsc.subcore_barrier()

    base = sid * STRIPE
    pltpu.sync_copy(s_stage.at[0, pl.ds(base, STRIPE)], accv)
    for t in range(1, NS):
        pltpu.sync_copy(s_stage.at[t, pl.ds(base, STRIPE)], tmpv)

        def _acc(i, carry):
            accv[pl.ds(i * L, L)] = (
                accv[pl.ds(i * L, L)] + tmpv[pl.ds(i * L, L)]
            )
            return carry

        lax.fori_loop(0, STRIPE // L, _acc, 0)
    pltpu.sync_copy(accv, s_out.at[cid, pl.ds(base, STRIPE)])


def _edge_softmax(src3, dst3, als, ald):
    return pl.kernel(
        _edge_softmax_body,
        out_type=[
            jax.ShapeDtypeStruct((NC, NPAD), jnp.float32),
            jax.ShapeDtypeStruct((NW, NB, B), jnp.float32),
        ],
        mesh=_mesh,
        scratch_types=[
            pltpu.VMEM((NPAD,), jnp.float32),
            pltpu.VMEM((NPAD,), jnp.float32),
            pltpu.VMEM((NB, B), jnp.int32),
            pltpu.VMEM((NB, B), jnp.int32),
            pltpu.VMEM((NB, B), jnp.float32),
            pltpu.VMEM((NPAD,), jnp.float32),
            pltpu.VMEM((STRIPE,), jnp.float32),
            pltpu.VMEM((STRIPE,), jnp.float32),
            pltpu.VMEM_SHARED((NS, NPAD), jnp.float32),
        ],
        compiler_params=pltpu.CompilerParams(needs_layout_passes=False),
    )(src3, dst3, als, ald)


def _aggregate_body(src_h, dst_h, ex_h, s_h, feat_h, out_h,
                    s_comb, tmp_s, srcb, exb, dst_row, alpha_v, rows,
                    out_sh, sem, sem2):
    cid = lax.axis_index("c")
    sid = lax.axis_index("s")
    wid = cid * NS + sid
    pltpu.sync_copy(s_h.at[0], s_comb)

    for c in range(NPAD // STRIPE):
        pltpu.sync_copy(s_h.at[1, pl.ds(c * STRIPE, STRIPE)], tmp_s)

        def _add(i, carry):
            s_comb[pl.ds(c * STRIPE + i * L, L)] = (
                s_comb[pl.ds(c * STRIPE + i * L, L)] + tmp_s[pl.ds(i * L, L)]
            )
            return carry

        lax.fori_loop(0, STRIPE // L, _add, 0)
    pltpu.sync_copy(src_h.at[wid], srcb)
    pltpu.sync_copy(ex_h.at[wid], exb)

    z16 = jnp.zeros((L,), jnp.float32)

    def _zero(i, carry):
        for k in range(D // L):
            rows[i, pl.ds(k * L, L)] = z16
        return carry

    lax.fori_loop(0, B, _zero, 0)
    for i in range(STRIPE // B):
        pltpu.sync_copy(rows, out_sh.at[pl.ds(sid * STRIPE + i * B, B)])
    plsc.subcore_barrier()

    def _row(j, carry):
        cp_d = pltpu.async_copy(dst_h.at[wid, j], dst_row, sem2)
        cp_r = pltpu.async_copy(feat_h.at[srcb.at[pl.ds(j * B, B)]], rows, sem)
        cp_d.wait()
        for k in range(B // L):
            d16 = dst_row[pl.ds(k * L, L)]
            s16 = plsc.load_gather(s_comb, [d16])
            ex16 = exb[pl.ds(j * B + k * L, L)]
            alpha_v[pl.ds(k * L, L)] = ex16 / (s16 + 1e-16)
        cp_r.wait()

        def _edge(e2, c2):
            asp = plsc.load_gather(alpha_v, [jnp.zeros((L,), jnp.int32) + e2])
            for k in range(D // L):
                rows[e2, pl.ds(k * L, L)] = rows[e2, pl.ds(k * L, L)] * asp
            return c2

        lax.fori_loop(0, B, _edge, 0)
        pltpu.sync_copy(rows, out_sh.at[dst_row], add=True)
        return carry

    lax.fori_loop(0, NB, _row, 0)
    plsc.subcore_barrier()
    pltpu.sync_copy(
        out_sh.at[pl.ds(sid * STRIPE, STRIPE)],
        out_h.at[cid, pl.ds(sid * STRIPE, STRIPE)],
    )


def _aggregate(src2, dst3, ex2, s_parts, feat):
    return pl.kernel(
        _aggregate_body,
        out_type=jax.ShapeDtypeStruct((NC, NPAD, D), jnp.float32),
        mesh=_mesh,
        scratch_types=[
            pltpu.VMEM((NPAD,), jnp.float32),
            pltpu.VMEM((STRIPE,), jnp.float32),
            pltpu.VMEM((EPT,), jnp.int32),
            pltpu.VMEM((EPT,), jnp.float32),
            pltpu.VMEM((B,), jnp.int32),
            pltpu.VMEM((B,), jnp.float32),
            pltpu.VMEM((B, D), jnp.float32),
            pltpu.VMEM_SHARED((NPAD, D), jnp.float32),
            pltpu.SemaphoreType.DMA,
            pltpu.SemaphoreType.DMA,
        ],
        compiler_params=pltpu.CompilerParams(needs_layout_passes=False),
    )(src2, dst3, ex2, s_parts, feat)


def kernel(x, edge_index, W1, a_src1, a_dst1, b1, g1, be1,
           W2, a_src2, a_dst2, b2, g2, be2):
    src3 = edge_index[0].reshape(NW, NB, B)
    dst3 = edge_index[1].reshape(NW, NB, B)
    src2 = edge_index[0].reshape(NW, EPT)
    zpad = jnp.zeros((NPAD - N,), jnp.float32)
    h1, als1, ald1 = _proj(x, W1, a_src1, a_dst1)
    s1, ex1 = _edge_softmax(src3, dst3,
                            jnp.concatenate([als1, zpad]),
                            jnp.concatenate([ald1, zpad]))
    p1 = _aggregate(src2, dst3, ex1.reshape(NW, EPT), s1, h1)
    h2, als2, ald2 = _mid(p1[0], p1[1], b1, g1, be1, W2, a_src2, a_dst2)
    s2, ex2 = _edge_softmax(src3, dst3,
                            jnp.concatenate([als2, zpad]),
                            jnp.concatenate([ald2, zpad]))
    p2 = _aggregate(src2, dst3, ex2.reshape(NW, EPT), s2, h2)
    return _final(p2[0], p2[1], b2, g2, be2)
